# baseline (speedup 1.0000x reference)
import functools
import math

import jax
import jax.numpy as jnp
import numpy as np
from jax import lax
from jax.experimental import pallas as pl
from jax.experimental.pallas import tpu as pltpu
from jax.experimental.pallas import tpu_sc as plsc

N_G = 16
R_CUT_CONST = 5.0
BLK = 1024
M_TOT = 16
K_CH = 32


def _sc_gather(table, idx, n_rows, d):
    info = plsc.get_sparse_core_info()
    nc, ns = info.num_cores, info.num_subcores
    nw = nc * ns
    per_w = n_rows // nw
    chunk = 1000
    n_chunks = per_w // chunk
    mesh = plsc.VectorSubcoreMesh(core_axis_name="c", subcore_axis_name="s")

    @functools.partial(
        pl.kernel, mesh=mesh,
        out_type=jax.ShapeDtypeStruct((n_rows, d), jnp.float32),
        compiler_params=pltpu.CompilerParams(use_tc_tiling_on_sc=False),
        scratch_types=[
            pltpu.VMEM((chunk,), jnp.int32),
            pltpu.VMEM((chunk,), jnp.int32),
            pltpu.VMEM((chunk, d), jnp.float32),
            pltpu.VMEM((chunk, d), jnp.float32),
            pltpu.SemaphoreType.DMA,
            pltpu.SemaphoreType.DMA,
        ],
    )
    def k(table_hbm, idx_hbm, out_hbm, idx_v0, idx_v1, rows_v0, rows_v1,
          gsem, osem):
        wid = lax.axis_index("s") * nc + lax.axis_index("c")
        base = wid * per_w
        bufs = ((idx_v0, rows_v0), (idx_v1, rows_v1))

        for j in range(n_chunks):
            idx_v, rows_v = bufs[j % 2]
            pltpu.sync_copy(idx_hbm.at[pl.ds(base + j * chunk, chunk)],
                            idx_v)
            gather = pltpu.async_copy(table_hbm.at[idx_v], rows_v, gsem)
            if j > 0:
                prev_rows = bufs[(j - 1) % 2][1]
                pltpu.async_copy(
                    prev_rows,
                    out_hbm.at[pl.ds(base + (j - 1) * chunk, chunk)],
                    osem).wait()
            gather.wait()
        last_rows = bufs[(n_chunks - 1) % 2][1]
        pltpu.async_copy(
            last_rows,
            out_hbm.at[pl.ds(base + (n_chunks - 1) * chunk, chunk)],
            osem).wait()

    return k(table, idx)


def _fused_kernel(idx_ref, r_ref, sh0_ref, sh1_ref, sh2_ref, sh3_ref,
                  emb_ref, w_ref, out_ref, src_ref, acc2_ref):
    i = pl.program_id(0)

    @pl.when(i == 0)
    def _():
        out_ref[...] = jnp.zeros_like(out_ref)
        acc2_ref[...] = jnp.zeros_like(acc2_ref)

    r = r_ref[:]
    mu = jax.lax.broadcasted_iota(jnp.int32, (1, N_G), 1).astype(
        jnp.float32) * (R_CUT_CONST / (N_G - 1))
    sigma = R_CUT_CONST / N_G
    g = jnp.exp(-0.5 * ((r[:, None] - mu) / sigma) ** 2)
    fc = 0.5 * (jnp.cos(jnp.pi * jnp.clip(r, 0.0, R_CUT_CONST) / R_CUT_CONST)
                + 1.0)
    gfc = g * fc[:, None]
    emb = emb_ref[:]

    cols = []
    for li, sh_ref in enumerate((sh0_ref, sh1_ref, sh2_ref, sh3_ref)):
        w = w_ref[li]
        rb = jax.lax.dot_general(gfc, w, (((1,), (0,)), ((), ())),
                                 preferred_element_type=jnp.float32)
        q = rb * emb
        sh = sh_ref[:]
        for m in range(2 * li + 1):
            cols.append(sh[:, m:m + 1] * q)
    src = jnp.concatenate(cols, axis=1)
    src_ref[...] = src

    def body(e, carry):
        c0 = idx_ref[2 * e]
        c1 = idx_ref[2 * e + 1]
        out_ref[pl.ds(c0, 1), :] = (out_ref[pl.ds(c0, 1), :]
                                    + src_ref[pl.ds(2 * e, 1), :])
        acc2_ref[pl.ds(c1, 1), :] = (acc2_ref[pl.ds(c1, 1), :]
                                     + src_ref[pl.ds(2 * e + 1, 1), :])
        return carry

    jax.lax.fori_loop(0, BLK // 2, body, 0, unroll=4)

    @pl.when(i == pl.num_programs(0) - 1)
    def _():
        out_ref[...] = out_ref[...] + acc2_ref[...]


def kernel(r, sh_l0, sh_l1, sh_l2, sh_l3, centers, neighbors, n_atoms,
           center_embedding, W0, W1, W2, W3):
    n_edges = r.shape[0]
    n_atoms_static = center_embedding.shape[0]
    k = K_CH
    grid = (n_edges // BLK,)
    emb_n = _sc_gather(center_embedding, neighbors.astype(jnp.int32),
                       n_edges, k)
    scatter_idx = (centers % n_atoms).astype(jnp.int32)
    w_all = jnp.stack([W0, W1, W2, W3])

    acc = pl.pallas_call(
        _fused_kernel,
        grid=grid,
        in_specs=[
            pl.BlockSpec((BLK,), lambda i: (i,), memory_space=pltpu.SMEM),
            pl.BlockSpec((BLK,), lambda i: (i,)),
            pl.BlockSpec((BLK, 1), lambda i: (i, 0)),
            pl.BlockSpec((BLK, 3), lambda i: (i, 0)),
            pl.BlockSpec((BLK, 5), lambda i: (i, 0)),
            pl.BlockSpec((BLK, 7), lambda i: (i, 0)),
            pl.BlockSpec((BLK, k), lambda i: (i, 0)),
            pl.BlockSpec((4, N_G, k), lambda i: (0, 0, 0)),
        ],
        out_specs=pl.BlockSpec((n_atoms_static, M_TOT * k), lambda i: (0, 0)),
        out_shape=jax.ShapeDtypeStruct((n_atoms_static, M_TOT * k),
                                       jnp.float32),
        scratch_shapes=[pltpu.VMEM((BLK, M_TOT * k), jnp.float32),
                        pltpu.VMEM((n_atoms_static, M_TOT * k), jnp.float32)],
    )(scatter_idx, r, sh_l0, sh_l1, sh_l2, sh_l3, emb_n, w_all)

    dens = acc.reshape(n_atoms_static, M_TOT, k)
    blocks = []
    off = 0
    for li in range(4):
        m = 2 * li + 1
        d = dens[:, off:off + m, :]
        off += m
        blocks.append(d / jnp.sqrt(jnp.mean(d * d) + 1e-10))
    return tuple(blocks)


# speedup vs baseline: 1.1154x; 1.1154x over previous
import functools
import math

import jax
import jax.numpy as jnp
import numpy as np
from jax import lax
from jax.experimental import pallas as pl
from jax.experimental.pallas import tpu as pltpu
from jax.experimental.pallas import tpu_sc as plsc

N_G = 16
R_CUT_CONST = 5.0
BLK = 1024
M_TOT = 16
K_CH = 32


def _sc_gather(table, idx, n_rows, d):
    info = plsc.get_sparse_core_info()
    nc, ns = info.num_cores, info.num_subcores
    nw = nc * ns
    per_w = n_rows // nw
    chunk = 1000
    n_chunks = per_w // chunk
    mesh = plsc.VectorSubcoreMesh(core_axis_name="c", subcore_axis_name="s")

    @functools.partial(
        pl.kernel, mesh=mesh,
        out_type=jax.ShapeDtypeStruct((n_rows, d), jnp.float32),
        compiler_params=pltpu.CompilerParams(use_tc_tiling_on_sc=False),
        scratch_types=[
            pltpu.VMEM((chunk,), jnp.int32),
            pltpu.VMEM((chunk,), jnp.int32),
            pltpu.VMEM((chunk, d), jnp.float32),
            pltpu.VMEM((chunk, d), jnp.float32),
            pltpu.SemaphoreType.DMA,
            pltpu.SemaphoreType.DMA,
        ],
    )
    def k(table_hbm, idx_hbm, out_hbm, idx_v0, idx_v1, rows_v0, rows_v1,
          gsem, osem):
        wid = lax.axis_index("s") * nc + lax.axis_index("c")
        base = wid * per_w
        bufs = ((idx_v0, rows_v0), (idx_v1, rows_v1))

        for j in range(n_chunks):
            idx_v, rows_v = bufs[j % 2]
            pltpu.sync_copy(idx_hbm.at[pl.ds(base + j * chunk, chunk)],
                            idx_v)
            gather = pltpu.async_copy(table_hbm.at[idx_v], rows_v, gsem)
            if j > 0:
                prev_rows = bufs[(j - 1) % 2][1]
                pltpu.async_copy(
                    prev_rows,
                    out_hbm.at[pl.ds(base + (j - 1) * chunk, chunk)],
                    osem).wait()
            gather.wait()
        last_rows = bufs[(n_chunks - 1) % 2][1]
        pltpu.async_copy(
            last_rows,
            out_hbm.at[pl.ds(base + (n_chunks - 1) * chunk, chunk)],
            osem).wait()

    return k(table, idx)


def _fused_kernel(idx_ref, r_ref, shc_ref, emb_ref, wc_ref, p_ref, q_ref,
                  out_ref, src_ref, acc2_ref):
    i = pl.program_id(0)

    @pl.when(i == 0)
    def _():
        out_ref[...] = jnp.zeros_like(out_ref)
        acc2_ref[...] = jnp.zeros_like(acc2_ref)

    r = r_ref[:]
    mu = jax.lax.broadcasted_iota(jnp.int32, (1, N_G), 1).astype(
        jnp.float32) * (R_CUT_CONST / (N_G - 1))
    sigma = R_CUT_CONST / N_G
    g = jnp.exp(-0.5 * ((r[:, None] - mu) / sigma) ** 2)
    fc = 0.5 * (jnp.cos(jnp.pi * jnp.clip(r, 0.0, R_CUT_CONST) / R_CUT_CONST)
                + 1.0)
    gfc = g * fc[:, None]

    rb_all = jax.lax.dot_general(gfc, wc_ref[...], (((1,), (0,)), ((), ())),
                                 preferred_element_type=jnp.float32)
    emb4 = jnp.tile(emb_ref[:], (1, 4))
    qc = rb_all * emb4
    q_exp = jax.lax.dot_general(qc, q_ref[...], (((1,), (0,)), ((), ())),
                                precision=jax.lax.Precision.HIGHEST,
                                preferred_element_type=jnp.float32)
    sh_exp = jax.lax.dot_general(shc_ref[:], p_ref[...],
                                 (((1,), (0,)), ((), ())),
                                 precision=jax.lax.Precision.HIGHEST,
                                 preferred_element_type=jnp.float32)
    src_ref[...] = sh_exp * q_exp

    def body(e, carry):
        c0 = idx_ref[2 * e]
        c1 = idx_ref[2 * e + 1]
        out_ref[pl.ds(c0, 1), :] = (out_ref[pl.ds(c0, 1), :]
                                    + src_ref[pl.ds(2 * e, 1), :])
        acc2_ref[pl.ds(c1, 1), :] = (acc2_ref[pl.ds(c1, 1), :]
                                     + src_ref[pl.ds(2 * e + 1, 1), :])
        return carry

    jax.lax.fori_loop(0, BLK // 2, body, 0, unroll=4)

    @pl.when(i == pl.num_programs(0) - 1)
    def _():
        out_ref[...] = out_ref[...] + acc2_ref[...]


def kernel(r, sh_l0, sh_l1, sh_l2, sh_l3, centers, neighbors, n_atoms,
           center_embedding, W0, W1, W2, W3):
    n_edges = r.shape[0]
    n_atoms_static = center_embedding.shape[0]
    k = K_CH
    grid = (n_edges // BLK,)
    emb_n = _sc_gather(center_embedding, neighbors.astype(jnp.int32),
                       n_edges, k)
    scatter_idx = (centers % n_atoms).astype(jnp.int32)
    sh_cat = jnp.concatenate([sh_l0, sh_l1, sh_l2, sh_l3], axis=1)
    w_cat = jnp.concatenate([W0, W1, W2, W3], axis=1)
    p_mat = jnp.asarray(np.kron(np.eye(M_TOT, dtype=np.float32),
                                np.ones((1, K_CH), np.float32)))
    l_of_m = [0] + [1] * 3 + [2] * 5 + [3] * 7
    q_np = np.zeros((4 * K_CH, M_TOT * K_CH), np.float32)
    for mg in range(M_TOT):
        li = l_of_m[mg]
        for kk in range(K_CH):
            q_np[li * K_CH + kk, mg * K_CH + kk] = 1.0
    q_mat = jnp.asarray(q_np)

    acc = pl.pallas_call(
        _fused_kernel,
        grid=grid,
        in_specs=[
            pl.BlockSpec((BLK,), lambda i: (i,), memory_space=pltpu.SMEM),
            pl.BlockSpec((BLK,), lambda i: (i,)),
            pl.BlockSpec((BLK, M_TOT), lambda i: (i, 0)),
            pl.BlockSpec((BLK, k), lambda i: (i, 0)),
            pl.BlockSpec((N_G, 4 * k), lambda i: (0, 0)),
            pl.BlockSpec((M_TOT, M_TOT * k), lambda i: (0, 0)),
            pl.BlockSpec((4 * k, M_TOT * k), lambda i: (0, 0)),
        ],
        out_specs=pl.BlockSpec((n_atoms_static, M_TOT * k), lambda i: (0, 0)),
        out_shape=jax.ShapeDtypeStruct((n_atoms_static, M_TOT * k),
                                       jnp.float32),
        scratch_shapes=[pltpu.VMEM((BLK, M_TOT * k), jnp.float32),
                        pltpu.VMEM((n_atoms_static, M_TOT * k), jnp.float32)],
    )(scatter_idx, r, sh_cat, emb_n, w_cat, p_mat, q_mat)

    dens = acc.reshape(n_atoms_static, M_TOT, k)
    blocks = []
    off = 0
    for li in range(4):
        m = 2 * li + 1
        d = dens[:, off:off + m, :]
        off += m
        blocks.append(d / jnp.sqrt(jnp.mean(d * d) + 1e-10))
    return tuple(blocks)


# speedup vs baseline: 1.1490x; 1.0301x over previous
import functools
import math

import jax
import jax.numpy as jnp
import numpy as np
from jax import lax
from jax.experimental import pallas as pl
from jax.experimental.pallas import tpu as pltpu
from jax.experimental.pallas import tpu_sc as plsc

N_G = 16
R_CUT_CONST = 5.0
BLK = 1024
M_TOT = 16
K_CH = 32


def _sc_gather(table, idx, n_rows, d):
    info = plsc.get_sparse_core_info()
    nc, ns = info.num_cores, info.num_subcores
    nw = nc * ns
    per_w = n_rows // nw
    chunk = 1000
    n_chunks = per_w // chunk
    mesh = plsc.VectorSubcoreMesh(core_axis_name="c", subcore_axis_name="s")

    @functools.partial(
        pl.kernel, mesh=mesh,
        out_type=jax.ShapeDtypeStruct((n_rows, d), jnp.float32),
        compiler_params=pltpu.CompilerParams(use_tc_tiling_on_sc=False),
        scratch_types=[
            pltpu.VMEM((chunk,), jnp.int32),
            pltpu.VMEM((chunk,), jnp.int32),
            pltpu.VMEM((chunk, d), jnp.float32),
            pltpu.VMEM((chunk, d), jnp.float32),
            pltpu.SemaphoreType.DMA,
            pltpu.SemaphoreType.DMA,
        ],
    )
    def k(table_hbm, idx_hbm, out_hbm, idx_v0, idx_v1, rows_v0, rows_v1,
          gsem, osem):
        wid = lax.axis_index("s") * nc + lax.axis_index("c")
        base = wid * per_w
        bufs = ((idx_v0, rows_v0), (idx_v1, rows_v1))

        for j in range(n_chunks):
            idx_v, rows_v = bufs[j % 2]
            pltpu.sync_copy(idx_hbm.at[pl.ds(base + j * chunk, chunk)],
                            idx_v)
            gather = pltpu.async_copy(table_hbm.at[idx_v], rows_v, gsem)
            if j > 0:
                prev_rows = bufs[(j - 1) % 2][1]
                pltpu.async_copy(
                    prev_rows,
                    out_hbm.at[pl.ds(base + (j - 1) * chunk, chunk)],
                    osem).wait()
            gather.wait()
        last_rows = bufs[(n_chunks - 1) % 2][1]
        pltpu.async_copy(
            last_rows,
            out_hbm.at[pl.ds(base + (n_chunks - 1) * chunk, chunk)],
            osem).wait()

    return k(table, idx)


def _fused_kernel(idx_ref, r_ref, shc_ref, emb_ref, wc_ref, p_ref, q_ref,
                  out_ref, src_ref, acc2_ref):
    i = pl.program_id(0)

    @pl.when(i == 0)
    def _():
        out_ref[...] = jnp.zeros_like(out_ref)
        acc2_ref[...] = jnp.zeros_like(acc2_ref)

    r = r_ref[:]
    mu = jax.lax.broadcasted_iota(jnp.int32, (1, N_G), 1).astype(
        jnp.float32) * (R_CUT_CONST / (N_G - 1))
    sigma = R_CUT_CONST / N_G
    g = jnp.exp(-0.5 * ((r[:, None] - mu) / sigma) ** 2)
    fc = 0.5 * (jnp.cos(jnp.pi * jnp.clip(r, 0.0, R_CUT_CONST) / R_CUT_CONST)
                + 1.0)
    gfc = g * fc[:, None]

    rb_all = jax.lax.dot_general(gfc, wc_ref[...], (((1,), (0,)), ((), ())),
                                 preferred_element_type=jnp.float32)
    emb4 = jnp.tile(emb_ref[:], (1, 4))
    qc = rb_all * emb4
    q_exp = jax.lax.dot_general(qc, q_ref[...], (((1,), (0,)), ((), ())),
                                precision=jax.lax.Precision.HIGHEST,
                                preferred_element_type=jnp.float32)
    sh_exp = jax.lax.dot_general(shc_ref[:], p_ref[...],
                                 (((1,), (0,)), ((), ())),
                                 precision=jax.lax.Precision.HIGHEST,
                                 preferred_element_type=jnp.float32)
    src_ref[...] = sh_exp * q_exp

    def body(e, carry):
        c0 = idx_ref[2 * e]
        c1 = idx_ref[2 * e + 1]
        out_ref[pl.ds(c0, 1), :] = (out_ref[pl.ds(c0, 1), :]
                                    + src_ref[pl.ds(2 * e, 1), :])
        acc2_ref[pl.ds(c1, 1), :] = (acc2_ref[pl.ds(c1, 1), :]
                                     + src_ref[pl.ds(2 * e + 1, 1), :])
        return carry

    jax.lax.fori_loop(0, BLK // 2, body, 0, unroll=8)

    @pl.when(i == pl.num_programs(0) - 1)
    def _():
        out_ref[...] = out_ref[...] + acc2_ref[...]


def kernel(r, sh_l0, sh_l1, sh_l2, sh_l3, centers, neighbors, n_atoms,
           center_embedding, W0, W1, W2, W3):
    n_edges = r.shape[0]
    n_atoms_static = center_embedding.shape[0]
    k = K_CH
    grid = (n_edges // BLK,)
    emb_n = _sc_gather(center_embedding, neighbors.astype(jnp.int32),
                       n_edges, k)
    scatter_idx = (centers % n_atoms).astype(jnp.int32)
    sh_cat = jnp.concatenate([sh_l0, sh_l1, sh_l2, sh_l3], axis=1)
    w_cat = jnp.concatenate([W0, W1, W2, W3], axis=1)
    p_mat = jnp.asarray(np.kron(np.eye(M_TOT, dtype=np.float32),
                                np.ones((1, K_CH), np.float32)))
    l_of_m = [0] + [1] * 3 + [2] * 5 + [3] * 7
    q_np = np.zeros((4 * K_CH, M_TOT * K_CH), np.float32)
    for mg in range(M_TOT):
        li = l_of_m[mg]
        for kk in range(K_CH):
            q_np[li * K_CH + kk, mg * K_CH + kk] = 1.0
    q_mat = jnp.asarray(q_np)

    acc = pl.pallas_call(
        _fused_kernel,
        grid=grid,
        in_specs=[
            pl.BlockSpec((BLK,), lambda i: (i,), memory_space=pltpu.SMEM),
            pl.BlockSpec((BLK,), lambda i: (i,)),
            pl.BlockSpec((BLK, M_TOT), lambda i: (i, 0)),
            pl.BlockSpec((BLK, k), lambda i: (i, 0)),
            pl.BlockSpec((N_G, 4 * k), lambda i: (0, 0)),
            pl.BlockSpec((M_TOT, M_TOT * k), lambda i: (0, 0)),
            pl.BlockSpec((4 * k, M_TOT * k), lambda i: (0, 0)),
        ],
        out_specs=pl.BlockSpec((n_atoms_static, M_TOT * k), lambda i: (0, 0)),
        out_shape=jax.ShapeDtypeStruct((n_atoms_static, M_TOT * k),
                                       jnp.float32),
        scratch_shapes=[pltpu.VMEM((BLK, M_TOT * k), jnp.float32),
                        pltpu.VMEM((n_atoms_static, M_TOT * k), jnp.float32)],
    )(scatter_idx, r, sh_cat, emb_n, w_cat, p_mat, q_mat)

    dens = acc.reshape(n_atoms_static, M_TOT, k)
    blocks = []
    off = 0
    for li in range(4):
        m = 2 * li + 1
        d = dens[:, off:off + m, :]
        off += m
        blocks.append(d / jnp.sqrt(jnp.mean(d * d) + 1e-10))
    return tuple(blocks)


# speedup vs baseline: 1.1495x; 1.0005x over previous
import functools

import jax
import jax.numpy as jnp
import numpy as np
from jax import lax
from jax.experimental import pallas as pl
from jax.experimental.pallas import tpu as pltpu
from jax.experimental.pallas import tpu_sc as plsc

N_G = 16
R_CUT_CONST = 5.0
BLK = 1024
M_TOT = 16
K_CH = 32


def _sc_gather(table, idx, n_rows, d):
    info = plsc.get_sparse_core_info()
    nc, ns = info.num_cores, info.num_subcores
    nw = nc * ns
    per_w = n_rows // nw
    chunk = 1000
    n_chunks = per_w // chunk
    mesh = plsc.VectorSubcoreMesh(core_axis_name="c", subcore_axis_name="s")

    @functools.partial(
        pl.kernel, mesh=mesh,
        out_type=jax.ShapeDtypeStruct((n_rows, d), jnp.float32),
        compiler_params=pltpu.CompilerParams(use_tc_tiling_on_sc=False),
        scratch_types=[
            pltpu.VMEM((chunk,), jnp.int32),
            pltpu.VMEM((chunk,), jnp.int32),
            pltpu.VMEM((chunk, d), jnp.float32),
            pltpu.VMEM((chunk, d), jnp.float32),
            pltpu.SemaphoreType.DMA,
            pltpu.SemaphoreType.DMA,
        ],
    )
    def k(table_hbm, idx_hbm, out_hbm, idx_v0, idx_v1, rows_v0, rows_v1,
          gsem, osem):
        wid = lax.axis_index("s") * nc + lax.axis_index("c")
        base = wid * per_w
        bufs = ((idx_v0, rows_v0), (idx_v1, rows_v1))

        for j in range(n_chunks):
            idx_v, rows_v = bufs[j % 2]
            pltpu.sync_copy(idx_hbm.at[pl.ds(base + j * chunk, chunk)],
                            idx_v)
            gather = pltpu.async_copy(table_hbm.at[idx_v], rows_v, gsem)
            if j > 0:
                prev_rows = bufs[(j - 1) % 2][1]
                pltpu.async_copy(
                    prev_rows,
                    out_hbm.at[pl.ds(base + (j - 1) * chunk, chunk)],
                    osem).wait()
            gather.wait()
        last_rows = bufs[(n_chunks - 1) % 2][1]
        pltpu.async_copy(
            last_rows,
            out_hbm.at[pl.ds(base + (n_chunks - 1) * chunk, chunk)],
            osem).wait()

    return k(table, idx)


def _fused_kernel(idx_ref, r_ref, shc_ref, emb_ref, wc_ref, p_ref, q_ref,
                  out_ref, src_ref, acc2_ref):
    i = pl.program_id(0)

    @pl.when(i == 0)
    def _():
        out_ref[...] = jnp.zeros_like(out_ref)
        acc2_ref[...] = jnp.zeros_like(acc2_ref)

    r = r_ref[:]
    mu = jax.lax.broadcasted_iota(jnp.int32, (1, N_G), 1).astype(
        jnp.float32) * (R_CUT_CONST / (N_G - 1))
    sigma = R_CUT_CONST / N_G
    g = jnp.exp(-0.5 * ((r[:, None] - mu) / sigma) ** 2)
    fc = 0.5 * (jnp.cos(jnp.pi * jnp.clip(r, 0.0, R_CUT_CONST) / R_CUT_CONST)
                + 1.0)
    gfc = g * fc[:, None]

    rb_all = jax.lax.dot_general(gfc, wc_ref[...], (((1,), (0,)), ((), ())),
                                 preferred_element_type=jnp.float32)
    emb4 = jnp.tile(emb_ref[:], (1, 4))
    qc = rb_all * emb4
    q_exp = jax.lax.dot_general(qc, q_ref[...], (((1,), (0,)), ((), ())),
                                precision=jax.lax.Precision.HIGHEST,
                                preferred_element_type=jnp.float32)
    sh_exp = jax.lax.dot_general(shc_ref[:], p_ref[...],
                                 (((1,), (0,)), ((), ())),
                                 precision=jax.lax.Precision.HIGHEST,
                                 preferred_element_type=jnp.float32)
    src_ref[...] = sh_exp * q_exp

    def body(e, carry):
        c0 = idx_ref[2 * e]
        c1 = idx_ref[2 * e + 1]
        out_ref[pl.ds(c0, 1), :] = (out_ref[pl.ds(c0, 1), :]
                                    + src_ref[pl.ds(2 * e, 1), :])
        acc2_ref[pl.ds(c1, 1), :] = (acc2_ref[pl.ds(c1, 1), :]
                                     + src_ref[pl.ds(2 * e + 1, 1), :])
        return carry

    jax.lax.fori_loop(0, BLK // 2, body, 0, unroll=8)

    @pl.when(i == pl.num_programs(0) - 1)
    def _():
        out_ref[...] = out_ref[...] + acc2_ref[...]


def kernel(r, sh_l0, sh_l1, sh_l2, sh_l3, centers, neighbors, n_atoms,
           center_embedding, W0, W1, W2, W3):
    n_edges = r.shape[0]
    n_atoms_static = center_embedding.shape[0]
    k = K_CH
    grid = (n_edges // BLK,)
    emb_n = _sc_gather(center_embedding, neighbors.astype(jnp.int32),
                       n_edges, k)
    scatter_idx = (centers % n_atoms).astype(jnp.int32)
    sh_cat = jnp.concatenate([sh_l0, sh_l1, sh_l2, sh_l3], axis=1)
    w_cat = jnp.concatenate([W0, W1, W2, W3], axis=1)
    p_mat = jnp.asarray(np.kron(np.eye(M_TOT, dtype=np.float32),
                                np.ones((1, K_CH), np.float32)))
    l_of_m = [0] + [1] * 3 + [2] * 5 + [3] * 7
    q_np = np.zeros((4 * K_CH, M_TOT * K_CH), np.float32)
    for mg in range(M_TOT):
        li = l_of_m[mg]
        for kk in range(K_CH):
            q_np[li * K_CH + kk, mg * K_CH + kk] = 1.0
    q_mat = jnp.asarray(q_np)

    acc = pl.pallas_call(
        _fused_kernel,
        grid=grid,
        in_specs=[
            pl.BlockSpec((BLK,), lambda i: (i,), memory_space=pltpu.SMEM),
            pl.BlockSpec((BLK,), lambda i: (i,)),
            pl.BlockSpec((BLK, M_TOT), lambda i: (i, 0)),
            pl.BlockSpec((BLK, k), lambda i: (i, 0)),
            pl.BlockSpec((N_G, 4 * k), lambda i: (0, 0)),
            pl.BlockSpec((M_TOT, M_TOT * k), lambda i: (0, 0)),
            pl.BlockSpec((4 * k, M_TOT * k), lambda i: (0, 0)),
        ],
        out_specs=pl.BlockSpec((n_atoms_static, M_TOT * k), lambda i: (0, 0)),
        out_shape=jax.ShapeDtypeStruct((n_atoms_static, M_TOT * k),
                                       jnp.float32),
        scratch_shapes=[pltpu.VMEM((BLK, M_TOT * k), jnp.float32),
                        pltpu.VMEM((n_atoms_static, M_TOT * k), jnp.float32)],
    )(scatter_idx, r, sh_cat, emb_n, w_cat, p_mat, q_mat)

    dens = acc.reshape(n_atoms_static, M_TOT, k)
    blocks = []
    off = 0
    for li in range(4):
        m = 2 * li + 1
        d = dens[:, off:off + m, :]
        off += m
        blocks.append(d / jnp.sqrt(jnp.mean(d * d) + 1e-10))
    return tuple(blocks)
